# baseline (speedup 1.0000x reference)
import functools

import jax
import jax.numpy as jnp
from jax import lax
from jax.experimental import pallas as pl
from jax.experimental.pallas import tpu as pltpu
from jax.experimental.pallas import tpu_sc as plsc

_D1 = 64
_D2 = 64
_M = _D1 + _D2
_DIM = _M * (_M + 1) // 2
_B = 16

_NS = 16
_RUNS_PER_W = _D1 // _NS
_LAST_RUN_END = _D2 + (_M - 1) * (_D1 - 1) - (_D1 - 1) * (_D1 - 2) // 2 + _D2
_TAIL_BLOCKS = -(-(_DIM - _LAST_RUN_END) // _D2)
_TAIL_PER_W = -(-_TAIL_BLOCKS // _NS)


def _s_of(i):
    return _D2 + (_M - 1) * i - (i * (i - 1)) // 2


@functools.cache
def _runcopy_kernel():
    mesh = plsc.VectorSubcoreMesh(
        core_axis_name="c", subcore_axis_name="s", num_cores=1
    )

    @functools.partial(
        pl.kernel,
        mesh=mesh,
        compiler_params=pltpu.CompilerParams(use_tc_tiling_on_sc=False),
        out_type=jax.ShapeDtypeStruct((_DIM, _B), jnp.float32),
        scratch_types=[
            pltpu.VMEM((_D2, _B), jnp.float32),
            pltpu.SemaphoreType.DMA,
        ],
    )
    def _body(x_hbm, out_hbm, zbuf_v, sem):
        wid = lax.axis_index("s")

        zero = jnp.zeros((_B,), jnp.float32)
        for j in range(_D2):
            zbuf_v[j] = zero

        zero_copies = []
        for k in range(_RUNS_PER_W):
            i = wid * _RUNS_PER_W + k
            zero_copies.append(
                pltpu.async_copy(
                    zbuf_v, out_hbm.at[pl.ds(_s_of(i) + _D2, _D2)], sem
                )
            )
        for k in range(_TAIL_PER_W):
            t = wid * _TAIL_PER_W + k
            off = jnp.minimum(_LAST_RUN_END + _D2 * t, _DIM - _D2)
            zero_copies.append(
                pltpu.async_copy(zbuf_v, out_hbm.at[pl.ds(off, _D2)], sem)
            )

        @pl.when(wid == 0)
        def _head_block():
            pltpu.sync_copy(zbuf_v, out_hbm.at[pl.ds(0, _D2)])

        for c in zero_copies:
            c.wait()
        plsc.subcore_barrier()

        run_copies = []
        for k in range(_RUNS_PER_W):
            i = wid * _RUNS_PER_W + k
            run_copies.append(
                pltpu.async_copy(
                    x_hbm.at[pl.ds(i * _D2, _D2)],
                    out_hbm.at[pl.ds(_s_of(i), _D2)],
                    sem,
                )
            )
        for c in run_copies:
            c.wait()

    return _body


def kernel(input_state, Passage_matrix):
    del Passage_matrix
    return _runcopy_kernel()(input_state)
